# baseline (speedup 1.0000x reference)
import functools

import jax
import jax.numpy as jnp
from jax import lax
from jax.experimental import pallas as pl
from jax.experimental.pallas import tpu as pltpu
from jax.experimental.pallas import tpu_sc as plsc

N = 10000
E = 320000
D = 128
P = 3

NPAD = 10048
NW = 32
CHUNK = 128
CH = 80
EPAD = NW * CH * CHUNK
STRIPE = NPAD // 16
GRP = CHUNK // 16

_mesh = plsc.VectorSubcoreMesh(core_axis_name="c", subcore_axis_name="s")


@functools.partial(
    pl.kernel,
    mesh=_mesh,
    out_type=(
        jax.ShapeDtypeStruct((2 * NPAD, D), jnp.float32),
        jax.ShapeDtypeStruct((NW, NPAD), jnp.float32),
    ),
    scratch_types=[
        pltpu.VMEM((CH, CHUNK), jnp.int32),
        pltpu.VMEM((CH, CHUNK), jnp.int32),
        pltpu.VMEM((CHUNK, D), jnp.float32),
        pltpu.VMEM((1, NPAD), jnp.float32),
        pltpu.VMEM_SHARED((NPAD, D), jnp.float32),
        pltpu.SemaphoreType.DMA,
    ],
    compiler_params=pltpu.CompilerParams(use_tc_tiling_on_sc=False,
                                         needs_layout_passes=False),
)
def _sc_segsum(u_hbm, dst_hbm, src_hbm, ztbl_hbm, zcnt_hbm,
               acc_out, cnt_out, dst_v, src_v, rows, cnt_v, acc_sh, sg):
    c = lax.axis_index("c")
    s = lax.axis_index("s")
    w = c * 16 + s
    pltpu.sync_copy(ztbl_hbm.at[pl.ds(s * STRIPE, STRIPE)],
                    acc_sh.at[pl.ds(s * STRIPE, STRIPE)])
    pltpu.sync_copy(zcnt_hbm, cnt_v)
    pltpu.sync_copy(dst_hbm.at[w], dst_v)
    pltpu.sync_copy(src_hbm.at[w], src_v)
    plsc.subcore_barrier()

    ones16 = jnp.ones((16,), jnp.float32)
    zero16 = jnp.zeros((16,), jnp.int32)

    def cnt_body(j, carry):
        for g in range(GRP):
            sv = src_v[j, pl.ds(g * 16, 16)]
            plsc.addupdate_scatter(cnt_v, [zero16, sv], ones16)
        return carry

    lax.fori_loop(0, CH, cnt_body, 0)

    def dma_body(j, carry):
        pltpu.async_copy(u_hbm.at[dst_v.at[j]], rows, sg).wait()
        pltpu.sync_copy(rows, acc_sh.at[src_v.at[j]], add=True)
        return carry

    lax.fori_loop(0, CH, dma_body, 0)
    plsc.subcore_barrier()

    pltpu.sync_copy(acc_sh.at[pl.ds(s * STRIPE, STRIPE)],
                    acc_out.at[pl.ds(c * NPAD + s * STRIPE, STRIPE)])
    pltpu.sync_copy(cnt_v, cnt_out.at[pl.ds(w, 1)])


BLK = 1256


def _prep_body(x_ref, pos_ref, w2_ref, w3_ref, u_ref):
    u_ref[...] = (jnp.dot(x_ref[...], w2_ref[...],
                          preferred_element_type=jnp.float32)
                  + jnp.dot(pos_ref[...], w3_ref[...],
                            preferred_element_type=jnp.float32))


def _tc_prep(x, pos8, w2, w3p):
    full = lambda shape: pl.BlockSpec(shape, lambda i: (0,) * len(shape))
    return pl.pallas_call(
        _prep_body,
        grid=(NPAD // BLK,),
        in_specs=[
            pl.BlockSpec((BLK, D), lambda i: (i, 0)),
            pl.BlockSpec((BLK, 8), lambda i: (i, 0)),
            full((D, D)),
            full((8, D)),
        ],
        out_specs=pl.BlockSpec((BLK, D), lambda i: (i, 0)),
        out_shape=jax.ShapeDtypeStruct((NPAD, D), jnp.float32),
    )(x, pos8, w2, w3p)


def _combine_body(x_ref, u_ref, acc_ref, cnt_ref, w1_ref, wa1_ref, wa2_ref,
                  bmsg_ref, bagg_ref, out_ref):
    x = x_ref[...]
    acc = acc_ref[0] + acc_ref[1]
    cnt = jnp.sum(cnt_ref[...], axis=1)[:, None]
    maxc = jnp.maximum(cnt, 1.0)
    ind = (cnt > 0.0).astype(jnp.float32)
    aggr = (ind * (jnp.dot(x, w1_ref[...],
                           preferred_element_type=jnp.float32)
                   - u_ref[...] + bmsg_ref[...])
            + acc / maxc)
    out_ref[...] = (jnp.dot(x, wa1_ref[...],
                            preferred_element_type=jnp.float32)
                    + jnp.dot(aggr, wa2_ref[...],
                              preferred_element_type=jnp.float32)
                    + bagg_ref[...])


def _tc_combine(x, u, acc, cnt, w1, wa1, wa2, bmsg, bagg):
    full = lambda shape: pl.BlockSpec(shape, lambda i: (0,) * len(shape))
    return pl.pallas_call(
        _combine_body,
        grid=(NPAD // BLK,),
        in_specs=[
            pl.BlockSpec((BLK, D), lambda i: (i, 0)),
            pl.BlockSpec((BLK, D), lambda i: (i, 0)),
            pl.BlockSpec((2, BLK, D), lambda i: (0, i, 0)),
            pl.BlockSpec((BLK, NW), lambda i: (i, 0)),
            full((D, D)),
            full((D, D)),
            full((D, D)),
            full((1, D)),
            full((1, D)),
        ],
        out_specs=pl.BlockSpec((BLK, D), lambda i: (i, 0)),
        out_shape=jax.ShapeDtypeStruct((NPAD, D), jnp.float32),
    )(x, u, acc, cnt, w1, wa1, wa2, bmsg, bagg)


def kernel(x, edge_index, pos, W_msg, b_msg, W_agg, b_agg):
    src = edge_index[0].astype(jnp.int32)
    dst = edge_index[1].astype(jnp.int32)
    pad_idx = jnp.full((EPAD - E,), N, jnp.int32)
    src_p = jnp.concatenate([src, pad_idx]).reshape(NW, CH, CHUNK)
    dst_p = jnp.concatenate([dst, pad_idx]).reshape(NW, CH, CHUNK)

    x_pad = jnp.zeros((NPAD, D), jnp.float32).at[:N].set(x)
    pos8 = jnp.zeros((NPAD, 8), jnp.float32).at[:N, :P].set(pos)

    W1, W2, W3 = W_msg[:D], W_msg[D:2 * D], W_msg[2 * D:]
    w3p = jnp.concatenate([W3, jnp.zeros((8 - P, D), jnp.float32)], axis=0)

    u = _tc_prep(x_pad, pos8, W2, w3p)

    ztbl = jnp.zeros((NPAD, D), jnp.float32)
    zcnt = jnp.zeros((1, NPAD), jnp.float32)
    acc, cnt = _sc_segsum(u, dst_p, src_p, ztbl, zcnt)
    acc = acc.reshape(2, NPAD, D)
    cnt = cnt.T

    out = _tc_combine(x_pad, u, acc, cnt, W1, W_agg[:D], W_agg[D:],
                      b_msg[None, :], b_agg[None, :])
    return out[:N]


# speedup vs baseline: 1.0098x; 1.0098x over previous
import functools

import jax
import jax.numpy as jnp
from jax import lax
from jax.experimental import pallas as pl
from jax.experimental.pallas import tpu as pltpu
from jax.experimental.pallas import tpu_sc as plsc

N = 10000
E = 320000
D = 128
P = 3

NPAD = 10048
NW = 32
CHUNK = 128
CH = 80
EPAD = NW * CH * CHUNK
STRIPE = NPAD // 16
GRP = CHUNK // 16

_mesh = plsc.VectorSubcoreMesh(core_axis_name="c", subcore_axis_name="s")


@functools.partial(
    pl.kernel,
    mesh=_mesh,
    out_type=(
        jax.ShapeDtypeStruct((2 * NPAD, D), jnp.float32),
        jax.ShapeDtypeStruct((NW, NPAD), jnp.float32),
    ),
    scratch_types=[
        pltpu.VMEM((CH, CHUNK), jnp.int32),
        pltpu.VMEM((CH, CHUNK), jnp.int32),
        pltpu.VMEM((CHUNK, D), jnp.float32),
        pltpu.VMEM((NPAD,), jnp.float32),
        pltpu.VMEM_SHARED((NPAD, D), jnp.float32),
        pltpu.SemaphoreType.DMA,
    ],
    compiler_params=pltpu.CompilerParams(use_tc_tiling_on_sc=False),
)
def _sc_segsum(u_hbm, dst_hbm, src_hbm, ztbl_hbm, zcnt_hbm,
               acc_out, cnt_out, dst_v, src_v, rows, cnt_v, acc_sh, sg):
    c = lax.axis_index("c")
    s = lax.axis_index("s")
    w = c * 16 + s
    pltpu.sync_copy(ztbl_hbm.at[pl.ds(s * STRIPE, STRIPE)],
                    acc_sh.at[pl.ds(s * STRIPE, STRIPE)])
    pltpu.sync_copy(zcnt_hbm, cnt_v)
    pltpu.sync_copy(dst_hbm.at[w], dst_v)
    pltpu.sync_copy(src_hbm.at[w], src_v)
    plsc.subcore_barrier()

    ones16 = jnp.ones((16,), jnp.float32)

    def cnt_body(j, carry):
        for g in range(GRP):
            sv = src_v[j, pl.ds(g * 16, 16)]
            cnt_v[pl.ds(0, 16)] = sv.astype(jnp.float32) + ones16
        return carry

    lax.fori_loop(0, CH, cnt_body, 0)

    def dma_body(j, carry):
        pltpu.async_copy(u_hbm.at[dst_v.at[j]], rows, sg).wait()
        pltpu.sync_copy(rows, acc_sh.at[src_v.at[j]], add=True)
        return carry

    lax.fori_loop(0, CH, dma_body, 0)
    plsc.subcore_barrier()

    pltpu.sync_copy(acc_sh.at[pl.ds(s * STRIPE, STRIPE)],
                    acc_out.at[pl.ds(c * NPAD + s * STRIPE, STRIPE)])
    pltpu.sync_copy(cnt_v, cnt_out.at[w])


BLK = 1256


def _prep_body(x_ref, pos_ref, w2_ref, w3_ref, u_ref):
    u_ref[...] = (jnp.dot(x_ref[...], w2_ref[...],
                          preferred_element_type=jnp.float32)
                  + jnp.dot(pos_ref[...], w3_ref[...],
                            preferred_element_type=jnp.float32))


def _tc_prep(x, pos8, w2, w3p):
    full = lambda shape: pl.BlockSpec(shape, lambda i: (0,) * len(shape))
    return pl.pallas_call(
        _prep_body,
        grid=(NPAD // BLK,),
        in_specs=[
            pl.BlockSpec((BLK, D), lambda i: (i, 0)),
            pl.BlockSpec((BLK, 8), lambda i: (i, 0)),
            full((D, D)),
            full((8, D)),
        ],
        out_specs=pl.BlockSpec((BLK, D), lambda i: (i, 0)),
        out_shape=jax.ShapeDtypeStruct((NPAD, D), jnp.float32),
    )(x, pos8, w2, w3p)


def _combine_body(x_ref, u_ref, acc_ref, cnt_ref, w1_ref, wa1_ref, wa2_ref,
                  bmsg_ref, bagg_ref, out_ref):
    x = x_ref[...]
    acc = acc_ref[0] + acc_ref[1]
    cnt = jnp.sum(cnt_ref[...], axis=1)[:, None]
    maxc = jnp.maximum(cnt, 1.0)
    ind = (cnt > 0.0).astype(jnp.float32)
    aggr = (ind * (jnp.dot(x, w1_ref[...],
                           preferred_element_type=jnp.float32)
                   - u_ref[...] + bmsg_ref[...])
            + acc / maxc)
    out_ref[...] = (jnp.dot(x, wa1_ref[...],
                            preferred_element_type=jnp.float32)
                    + jnp.dot(aggr, wa2_ref[...],
                              preferred_element_type=jnp.float32)
                    + bagg_ref[...])


def _tc_combine(x, u, acc, cnt, w1, wa1, wa2, bmsg, bagg):
    full = lambda shape: pl.BlockSpec(shape, lambda i: (0,) * len(shape))
    return pl.pallas_call(
        _combine_body,
        grid=(NPAD // BLK,),
        in_specs=[
            pl.BlockSpec((BLK, D), lambda i: (i, 0)),
            pl.BlockSpec((BLK, D), lambda i: (i, 0)),
            pl.BlockSpec((2, BLK, D), lambda i: (0, i, 0)),
            pl.BlockSpec((BLK, NW), lambda i: (i, 0)),
            full((D, D)),
            full((D, D)),
            full((D, D)),
            full((1, D)),
            full((1, D)),
        ],
        out_specs=pl.BlockSpec((BLK, D), lambda i: (i, 0)),
        out_shape=jax.ShapeDtypeStruct((NPAD, D), jnp.float32),
    )(x, u, acc, cnt, w1, wa1, wa2, bmsg, bagg)


def kernel(x, edge_index, pos, W_msg, b_msg, W_agg, b_agg):
    src = edge_index[0].astype(jnp.int32)
    dst = edge_index[1].astype(jnp.int32)
    pad_idx = jnp.full((EPAD - E,), N, jnp.int32)
    src_p = jnp.concatenate([src, pad_idx]).reshape(NW, CH, CHUNK)
    dst_p = jnp.concatenate([dst, pad_idx]).reshape(NW, CH, CHUNK)

    x_pad = jnp.zeros((NPAD, D), jnp.float32).at[:N].set(x)
    pos8 = jnp.zeros((NPAD, 8), jnp.float32).at[:N, :P].set(pos)

    W1, W2, W3 = W_msg[:D], W_msg[D:2 * D], W_msg[2 * D:]
    w3p = jnp.concatenate([W3, jnp.zeros((8 - P, D), jnp.float32)], axis=0)

    u = _tc_prep(x_pad, pos8, W2, w3p)

    ztbl = jnp.zeros((NPAD, D), jnp.float32)
    zcnt = jnp.zeros((NPAD,), jnp.float32)
    acc, cnt = _sc_segsum(u, dst_p, src_p, ztbl, zcnt)
    acc = acc.reshape(2, NPAD, D)
    cnt = cnt.T

    out = _tc_combine(x_pad, u, acc, cnt, W1, W_agg[:D], W_agg[D:],
                      b_msg[None, :], b_agg[None, :])
    return out[:N]


# speedup vs baseline: 1.0158x; 1.0059x over previous
import functools

import jax
import jax.numpy as jnp
from jax import lax
from jax.experimental import pallas as pl
from jax.experimental.pallas import tpu as pltpu
from jax.experimental.pallas import tpu_sc as plsc

N = 10000
E = 320000
D = 128
P = 3

NPAD = 10048
NW = 32
CHUNK = 128
CH = 80
EPAD = NW * CH * CHUNK
STRIPE = NPAD // 16
GRP = CHUNK // 16

_mesh = plsc.VectorSubcoreMesh(core_axis_name="c", subcore_axis_name="s")


@functools.partial(
    pl.kernel,
    mesh=_mesh,
    out_type=(
        jax.ShapeDtypeStruct((2 * NPAD, D), jnp.float32),
        jax.ShapeDtypeStruct((NW, NPAD), jnp.float32),
    ),
    scratch_types=[
        pltpu.VMEM((CH, CHUNK), jnp.int32),
        pltpu.VMEM((CH, CHUNK), jnp.int32),
        pltpu.VMEM((CHUNK, D), jnp.float32),
        pltpu.VMEM((NPAD,), jnp.float32),
        pltpu.VMEM_SHARED((NPAD, D), jnp.float32),
        pltpu.SemaphoreType.DMA,
    ],
    compiler_params=pltpu.CompilerParams(use_tc_tiling_on_sc=False),
)
def _sc_segsum(u_hbm, dst_hbm, src_hbm, ztbl_hbm, zcnt_hbm,
               acc_out, cnt_out, dst_v, src_v, rows, cnt_v, acc_sh, sg):
    c = lax.axis_index("c")
    s = lax.axis_index("s")
    w = c * 16 + s
    pltpu.sync_copy(ztbl_hbm.at[pl.ds(s * STRIPE, STRIPE)],
                    acc_sh.at[pl.ds(s * STRIPE, STRIPE)])
    pltpu.sync_copy(zcnt_hbm, cnt_v)
    pltpu.sync_copy(dst_hbm.at[w], dst_v)
    pltpu.sync_copy(src_hbm.at[w], src_v)
    plsc.subcore_barrier()

    ones16 = jnp.ones((16,), jnp.float32)


    def dma_body(j, carry):
        pltpu.async_copy(u_hbm.at[dst_v.at[j]], rows, sg).wait()
        pltpu.sync_copy(rows, acc_sh.at[src_v.at[j]], add=True)
        return carry

    lax.fori_loop(0, CH, dma_body, 0)
    plsc.subcore_barrier()

    pltpu.sync_copy(acc_sh.at[pl.ds(s * STRIPE, STRIPE)],
                    acc_out.at[pl.ds(c * NPAD + s * STRIPE, STRIPE)])
    pltpu.sync_copy(cnt_v, cnt_out.at[w])


BLK = 1256


def _prep_body(x_ref, pos_ref, w2_ref, w3_ref, u_ref):
    u_ref[...] = (jnp.dot(x_ref[...], w2_ref[...],
                          preferred_element_type=jnp.float32)
                  + jnp.dot(pos_ref[...], w3_ref[...],
                            preferred_element_type=jnp.float32))


def _tc_prep(x, pos8, w2, w3p):
    full = lambda shape: pl.BlockSpec(shape, lambda i: (0,) * len(shape))
    return pl.pallas_call(
        _prep_body,
        grid=(NPAD // BLK,),
        in_specs=[
            pl.BlockSpec((BLK, D), lambda i: (i, 0)),
            pl.BlockSpec((BLK, 8), lambda i: (i, 0)),
            full((D, D)),
            full((8, D)),
        ],
        out_specs=pl.BlockSpec((BLK, D), lambda i: (i, 0)),
        out_shape=jax.ShapeDtypeStruct((NPAD, D), jnp.float32),
    )(x, pos8, w2, w3p)


def _combine_body(x_ref, u_ref, acc_ref, cnt_ref, w1_ref, wa1_ref, wa2_ref,
                  bmsg_ref, bagg_ref, out_ref):
    x = x_ref[...]
    acc = acc_ref[0] + acc_ref[1]
    cnt = jnp.sum(cnt_ref[...], axis=1)[:, None]
    maxc = jnp.maximum(cnt, 1.0)
    ind = (cnt > 0.0).astype(jnp.float32)
    aggr = (ind * (jnp.dot(x, w1_ref[...],
                           preferred_element_type=jnp.float32)
                   - u_ref[...] + bmsg_ref[...])
            + acc / maxc)
    out_ref[...] = (jnp.dot(x, wa1_ref[...],
                            preferred_element_type=jnp.float32)
                    + jnp.dot(aggr, wa2_ref[...],
                              preferred_element_type=jnp.float32)
                    + bagg_ref[...])


def _tc_combine(x, u, acc, cnt, w1, wa1, wa2, bmsg, bagg):
    full = lambda shape: pl.BlockSpec(shape, lambda i: (0,) * len(shape))
    return pl.pallas_call(
        _combine_body,
        grid=(NPAD // BLK,),
        in_specs=[
            pl.BlockSpec((BLK, D), lambda i: (i, 0)),
            pl.BlockSpec((BLK, D), lambda i: (i, 0)),
            pl.BlockSpec((2, BLK, D), lambda i: (0, i, 0)),
            pl.BlockSpec((BLK, NW), lambda i: (i, 0)),
            full((D, D)),
            full((D, D)),
            full((D, D)),
            full((1, D)),
            full((1, D)),
        ],
        out_specs=pl.BlockSpec((BLK, D), lambda i: (i, 0)),
        out_shape=jax.ShapeDtypeStruct((NPAD, D), jnp.float32),
    )(x, u, acc, cnt, w1, wa1, wa2, bmsg, bagg)


def kernel(x, edge_index, pos, W_msg, b_msg, W_agg, b_agg):
    src = edge_index[0].astype(jnp.int32)
    dst = edge_index[1].astype(jnp.int32)
    pad_idx = jnp.full((EPAD - E,), N, jnp.int32)
    src_p = jnp.concatenate([src, pad_idx]).reshape(NW, CH, CHUNK)
    dst_p = jnp.concatenate([dst, pad_idx]).reshape(NW, CH, CHUNK)

    x_pad = jnp.zeros((NPAD, D), jnp.float32).at[:N].set(x)
    pos8 = jnp.zeros((NPAD, 8), jnp.float32).at[:N, :P].set(pos)

    W1, W2, W3 = W_msg[:D], W_msg[D:2 * D], W_msg[2 * D:]
    w3p = jnp.concatenate([W3, jnp.zeros((8 - P, D), jnp.float32)], axis=0)

    u = _tc_prep(x_pad, pos8, W2, w3p)

    ztbl = jnp.zeros((NPAD, D), jnp.float32)
    zcnt = jnp.zeros((NPAD,), jnp.float32)
    acc, cnt = _sc_segsum(u, dst_p, src_p, ztbl, zcnt)
    acc = acc.reshape(2, NPAD, D)
    cnt = cnt.T

    out = _tc_combine(x_pad, u, acc, cnt, W1, W_agg[:D], W_agg[D:],
                      b_msg[None, :], b_agg[None, :])
    return out[:N]


# speedup vs baseline: 1.4080x; 1.3862x over previous
import functools

import jax
import jax.numpy as jnp
from jax import lax
from jax.experimental import pallas as pl
from jax.experimental.pallas import tpu as pltpu
from jax.experimental.pallas import tpu_sc as plsc

N = 10000
E = 320000
D = 128
P = 3

DT = 144
NPAD = 10240
NW = 32
CHUNK = 128
CH = 79
EPAD = NW * CH * CHUNK
STRIPE = NPAD // 16

_mesh = plsc.VectorSubcoreMesh(core_axis_name="c", subcore_axis_name="s")


@functools.partial(
    pl.kernel,
    mesh=_mesh,
    out_type=jax.ShapeDtypeStruct((2 * NPAD, DT), jnp.float32),
    scratch_types=[
        pltpu.VMEM((CH, CHUNK), jnp.int32),
        pltpu.VMEM((CH, CHUNK), jnp.int32),
        pltpu.VMEM((CHUNK, DT), jnp.float32),
        pltpu.VMEM_SHARED((NPAD, DT), jnp.float32),
        pltpu.SemaphoreType.DMA,
    ],
    compiler_params=pltpu.CompilerParams(use_tc_tiling_on_sc=False),
)
def _sc_segsum(t_hbm, dst_hbm, src_hbm, zero_hbm, out_hbm,
               dst_v, src_v, rows_v, acc_sh, sem):
    c = lax.axis_index("c")
    s = lax.axis_index("s")
    w = c * 16 + s
    pltpu.sync_copy(zero_hbm.at[pl.ds(s * STRIPE, STRIPE)],
                    acc_sh.at[pl.ds(s * STRIPE, STRIPE)])
    pltpu.sync_copy(dst_hbm.at[w], dst_v)
    pltpu.sync_copy(src_hbm.at[w], src_v)
    plsc.subcore_barrier()

    def body(j, carry):
        pltpu.async_copy(t_hbm.at[dst_v.at[j]], rows_v, sem).wait()
        pltpu.sync_copy(rows_v, acc_sh.at[src_v.at[j]], add=True)
        return carry

    lax.fori_loop(0, CH, body, 0)
    plsc.subcore_barrier()
    pltpu.sync_copy(acc_sh.at[pl.ds(s * STRIPE, STRIPE)],
                    out_hbm.at[pl.ds(c * NPAD + s * STRIPE, STRIPE)])


BLK = 1024


def _tc_body(t_ref, acc_ref, walpha_ref, wbeta_ref, wa1_ref, wa2_ref,
             bagg_ref, out_ref):
    t = t_ref[...]
    acc = acc_ref[0] + acc_ref[1]
    cnt = acc[:, D + P:D + P + 1]
    maxc = jnp.maximum(cnt, 1.0)
    ind = (cnt > 0.0).astype(jnp.float32)
    aggr = (ind * jnp.dot(t, walpha_ref[...],
                          preferred_element_type=jnp.float32)
            + jnp.dot(acc / maxc, wbeta_ref[...],
                      preferred_element_type=jnp.float32))
    out_ref[...] = (jnp.dot(t[:, :D], wa1_ref[...],
                            preferred_element_type=jnp.float32)
                    + jnp.dot(aggr, wa2_ref[...],
                              preferred_element_type=jnp.float32)
                    + bagg_ref[...])


def _tc_combine(t, partials, walpha, wbeta, wa1, wa2, bagg):
    full = lambda shape: pl.BlockSpec(shape, lambda i: (0,) * len(shape))
    return pl.pallas_call(
        _tc_body,
        grid=(NPAD // BLK,),
        in_specs=[
            pl.BlockSpec((BLK, DT), lambda i: (i, 0)),
            pl.BlockSpec((2, BLK, DT), lambda i: (0, i, 0)),
            full((DT, D)),
            full((DT, D)),
            full((D, D)),
            full((D, D)),
            full((1, D)),
        ],
        out_specs=pl.BlockSpec((BLK, D), lambda i: (i, 0)),
        out_shape=jax.ShapeDtypeStruct((NPAD, D), jnp.float32),
    )(t, partials, walpha, wbeta, wa1, wa2, bagg)


def kernel(x, edge_index, pos, W_msg, b_msg, W_agg, b_agg):
    src = edge_index[0].astype(jnp.int32)
    dst = edge_index[1].astype(jnp.int32)
    npad_edges = EPAD - E
    pad_idx = jnp.full((npad_edges,), N, jnp.int32)
    src_p = jnp.concatenate([src, pad_idx]).reshape(NW, CH, CHUNK)
    dst_p = jnp.concatenate([dst, pad_idx]).reshape(NW, CH, CHUNK)

    t = jnp.zeros((NPAD, DT), jnp.float32)
    t = t.at[:N, :D].set(x).at[:N, D:D + P].set(pos).at[:N, D + P].set(1.0)
    zeros_tbl = jnp.zeros((NPAD, DT), jnp.float32)

    partials = _sc_segsum(t, dst_p, src_p, zeros_tbl).reshape(2, NPAD, DT)

    W1, W2, W3 = W_msg[:D], W_msg[D:2 * D], W_msg[2 * D:]
    zpad = jnp.zeros((DT - D - P - 1, D), jnp.float32)
    walpha = jnp.concatenate([W1 - W2, -W3, b_msg[None, :], zpad], axis=0)
    wbeta = jnp.concatenate([W2, W3, jnp.zeros((DT - D - P, D), jnp.float32)],
                            axis=0)

    out = _tc_combine(t, partials, walpha, wbeta,
                      W_agg[:D], W_agg[D:], b_agg[None, :])
    return out[:N]


# speedup vs baseline: 1.8801x; 1.3353x over previous
import functools

import jax
import jax.numpy as jnp
from jax import lax
from jax.experimental import pallas as pl
from jax.experimental.pallas import tpu as pltpu
from jax.experimental.pallas import tpu_sc as plsc

N = 10000
E = 320000
D = 128
P = 3

DT = 128
NPAD = 10240
NW = 32
CHUNK = 128
CH = 79
EPAD = NW * CH * CHUNK
STRIPE = NPAD // 16

_mesh = plsc.VectorSubcoreMesh(core_axis_name="c", subcore_axis_name="s")


@functools.partial(
    pl.kernel,
    mesh=_mesh,
    out_type=jax.ShapeDtypeStruct((2 * NPAD, DT), jnp.float32),
    scratch_types=[
        pltpu.VMEM((CH, CHUNK), jnp.int32),
        pltpu.VMEM((CH, CHUNK), jnp.int32),
        pltpu.VMEM((CHUNK, DT), jnp.float32),
        pltpu.VMEM_SHARED((NPAD, DT), jnp.float32),
        pltpu.SemaphoreType.DMA,
    ],
    compiler_params=pltpu.CompilerParams(use_tc_tiling_on_sc=False),
)
def _sc_segsum(t_hbm, dst_hbm, src_hbm, zero_hbm, out_hbm,
               dst_v, src_v, rows_v, acc_sh, sem):
    c = lax.axis_index("c")
    s = lax.axis_index("s")
    w = c * 16 + s
    pltpu.sync_copy(zero_hbm.at[pl.ds(s * STRIPE, STRIPE)],
                    acc_sh.at[pl.ds(s * STRIPE, STRIPE)])
    pltpu.sync_copy(dst_hbm.at[w], dst_v)
    pltpu.sync_copy(src_hbm.at[w], src_v)
    plsc.subcore_barrier()

    def body(j, carry):
        pltpu.async_copy(t_hbm.at[dst_v.at[j]], rows_v, sem).wait()
        pltpu.sync_copy(rows_v, acc_sh.at[src_v.at[j]], add=True)
        return carry

    lax.fori_loop(0, CH, body, 0)
    plsc.subcore_barrier()
    pltpu.sync_copy(acc_sh.at[pl.ds(s * STRIPE, STRIPE)],
                    out_hbm.at[pl.ds(c * NPAD + s * STRIPE, STRIPE)])


BLK = 1024


def _tc_body(t_ref, acc_ref, walpha_ref, wbeta_ref, wa1_ref, wa2_ref,
             bagg_ref, out_ref):
    t = t_ref[...]
    acc = acc_ref[0] + acc_ref[1]
    cnt = acc[:, D - 1:D]
    maxc = jnp.maximum(cnt, 1.0)
    ind = (cnt > 0.0).astype(jnp.float32)
    aggr = (ind * jnp.dot(t, walpha_ref[...],
                          preferred_element_type=jnp.float32)
            + jnp.dot(acc / maxc, wbeta_ref[...],
                      preferred_element_type=jnp.float32))
    out_ref[...] = (jnp.dot(t[:, :D], wa1_ref[...],
                            preferred_element_type=jnp.float32)
                    + jnp.dot(aggr, wa2_ref[...],
                              preferred_element_type=jnp.float32)
                    + bagg_ref[...])


def _tc_combine(t, partials, walpha, wbeta, wa1, wa2, bagg):
    full = lambda shape: pl.BlockSpec(shape, lambda i: (0,) * len(shape))
    return pl.pallas_call(
        _tc_body,
        grid=(NPAD // BLK,),
        in_specs=[
            pl.BlockSpec((BLK, DT), lambda i: (i, 0)),
            pl.BlockSpec((2, BLK, DT), lambda i: (0, i, 0)),
            full((DT, D)),
            full((DT, D)),
            full((D, D)),
            full((D, D)),
            full((1, D)),
        ],
        out_specs=pl.BlockSpec((BLK, D), lambda i: (i, 0)),
        out_shape=jax.ShapeDtypeStruct((NPAD, D), jnp.float32),
    )(t, partials, walpha, wbeta, wa1, wa2, bagg)


def kernel(x, edge_index, pos, W_msg, b_msg, W_agg, b_agg):
    src = edge_index[0].astype(jnp.int32)
    dst = edge_index[1].astype(jnp.int32)
    npad_edges = EPAD - E
    pad_idx = jnp.full((npad_edges,), N, jnp.int32)
    src_p = jnp.concatenate([src, pad_idx]).reshape(NW, CH, CHUNK)
    dst_p = jnp.concatenate([dst, pad_idx]).reshape(NW, CH, CHUNK)

    t = jnp.zeros((NPAD, DT), jnp.float32)
    t = t.at[:N, :D].set(x)
    zeros_tbl = jnp.zeros((NPAD, DT), jnp.float32)

    partials = _sc_segsum(t, dst_p, src_p, zeros_tbl).reshape(2, NPAD, DT)

    W1, W2, W3 = W_msg[:D], W_msg[D:2 * D], W_msg[2 * D:]
    walpha = W1 - W2
    wbeta = W2

    out = _tc_combine(t, partials, walpha, wbeta,
                      W_agg[:D], W_agg[D:], b_agg[None, :])
    return out[:N]


# speedup vs baseline: 1.9433x; 1.0336x over previous
import functools

import jax
import jax.numpy as jnp
from jax import lax
from jax.experimental import pallas as pl
from jax.experimental.pallas import tpu as pltpu
from jax.experimental.pallas import tpu_sc as plsc

N = 10000
E = 320000
D = 128
P = 3

DT = 144
NPAD = 10240
NW = 32
CHUNK = 128
CH = 79
EPAD = NW * CH * CHUNK
STRIPE = NPAD // 16

_mesh = plsc.VectorSubcoreMesh(core_axis_name="c", subcore_axis_name="s")


@functools.partial(
    pl.kernel,
    mesh=_mesh,
    out_type=jax.ShapeDtypeStruct((2 * NPAD, DT), jnp.float32),
    scratch_types=[
        pltpu.VMEM((CH, CHUNK), jnp.int32),
        pltpu.VMEM((CH, CHUNK), jnp.int32),
        pltpu.VMEM((CHUNK, DT), jnp.float32),
        pltpu.VMEM_SHARED((NPAD, DT), jnp.float32),
        pltpu.SemaphoreType.DMA,
    ],
    compiler_params=pltpu.CompilerParams(use_tc_tiling_on_sc=False),
)
def _sc_segsum(t_hbm, dst_hbm, src_hbm, zero_hbm, out_hbm,
               dst_v, src_v, rows_v, acc_sh, sem):
    c = lax.axis_index("c")
    s = lax.axis_index("s")
    w = c * 16 + s
    pltpu.sync_copy(zero_hbm.at[pl.ds(s * STRIPE, STRIPE)],
                    acc_sh.at[pl.ds(s * STRIPE, STRIPE)])
    pltpu.sync_copy(dst_hbm.at[w], dst_v)
    pltpu.sync_copy(src_hbm.at[w], src_v)
    plsc.subcore_barrier()

    def body(j, carry):
        pltpu.async_copy(t_hbm.at[dst_v.at[j]], rows_v, sem).wait()
        pltpu.sync_copy(rows_v, acc_sh.at[src_v.at[j]], add=True)
        return carry

    lax.fori_loop(0, CH, body, 0)
    plsc.subcore_barrier()
    pltpu.sync_copy(acc_sh.at[pl.ds(s * STRIPE, STRIPE)],
                    out_hbm.at[pl.ds(c * NPAD + s * STRIPE, STRIPE)])


BLK = 1024


def _tc_body(t_ref, acc_ref, walpha_ref, wbeta_ref, wa1_ref, wa2_ref,
             bagg_ref, out_ref):
    t = t_ref[...]
    acc = acc_ref[0] + acc_ref[1]
    cnt = acc[:, D + P:D + P + 1]
    maxc = jnp.maximum(cnt, 1.0)
    ind = (cnt > 0.0).astype(jnp.float32)
    aggr = (ind * jnp.dot(t, walpha_ref[...],
                          preferred_element_type=jnp.float32)
            + jnp.dot(acc / maxc, wbeta_ref[...],
                      preferred_element_type=jnp.float32))
    out_ref[...] = (jnp.dot(t[:, :D], wa1_ref[...],
                            preferred_element_type=jnp.float32)
                    + jnp.dot(aggr, wa2_ref[...],
                              preferred_element_type=jnp.float32)
                    + bagg_ref[...])


def _tc_combine(t, partials, walpha, wbeta, wa1, wa2, bagg):
    full = lambda shape: pl.BlockSpec(shape, lambda i: (0,) * len(shape))
    return pl.pallas_call(
        _tc_body,
        grid=(NPAD // BLK,),
        in_specs=[
            pl.BlockSpec((BLK, DT), lambda i: (i, 0)),
            pl.BlockSpec((2, BLK, DT), lambda i: (0, i, 0)),
            full((DT, D)),
            full((DT, D)),
            full((D, D)),
            full((D, D)),
            full((1, D)),
        ],
        out_specs=pl.BlockSpec((BLK, D), lambda i: (i, 0)),
        out_shape=jax.ShapeDtypeStruct((NPAD, D), jnp.float32),
    )(t, partials, walpha, wbeta, wa1, wa2, bagg)


def kernel(x, edge_index, pos, W_msg, b_msg, W_agg, b_agg):
    src = edge_index[0].astype(jnp.int32)
    dst = edge_index[1].astype(jnp.int32)
    ppw = CH * CHUNK - E // NW
    pad_rows = N + (jnp.arange(NW * ppw, dtype=jnp.int32) % (NPAD - N))
    pad_rows = pad_rows.reshape(NW, ppw)
    src_p = jnp.concatenate([src.reshape(NW, E // NW), pad_rows],
                            axis=1).reshape(NW, CH, CHUNK)
    dst_p = jnp.concatenate([dst.reshape(NW, E // NW), pad_rows],
                            axis=1).reshape(NW, CH, CHUNK)

    t = jnp.zeros((NPAD, DT), jnp.float32)
    t = t.at[:N, :D].set(x).at[:N, D:D + P].set(pos).at[:N, D + P].set(1.0)
    zeros_tbl = jnp.zeros((NPAD, DT), jnp.float32)

    partials = _sc_segsum(t, dst_p, src_p, zeros_tbl).reshape(2, NPAD, DT)

    W1, W2, W3 = W_msg[:D], W_msg[D:2 * D], W_msg[2 * D:]
    zpad = jnp.zeros((DT - D - P - 1, D), jnp.float32)
    walpha = jnp.concatenate([W1 - W2, -W3, b_msg[None, :], zpad], axis=0)
    wbeta = jnp.concatenate([W2, W3, jnp.zeros((DT - D - P, D), jnp.float32)],
                            axis=0)

    out = _tc_combine(t, partials, walpha, wbeta,
                      W_agg[:D], W_agg[D:], b_agg[None, :])
    return out[:N]


# speedup vs baseline: 2.5754x; 1.3253x over previous
import functools

import jax
import jax.numpy as jnp
from jax import lax
from jax.experimental import pallas as pl
from jax.experimental.pallas import tpu as pltpu
from jax.experimental.pallas import tpu_sc as plsc

N = 10000
E = 320000
D = 128
P = 3

NPAD = 10240
NW = 32
CHUNK = 128
CH = 79
EPAD = NW * CH * CHUNK
STRIPE = NPAD // 16
GRP = CHUNK // 16

_mesh = plsc.VectorSubcoreMesh(core_axis_name="c", subcore_axis_name="s")


@functools.partial(
    pl.kernel,
    mesh=_mesh,
    out_type=(
        jax.ShapeDtypeStruct((2 * NPAD, D), jnp.float32),
        jax.ShapeDtypeStruct((NW, NPAD), jnp.float32),
    ),
    scratch_types=[
        pltpu.VMEM((CH, CHUNK), jnp.int32),
        pltpu.VMEM((CH, CHUNK), jnp.int32),
        pltpu.VMEM((CHUNK, D), jnp.float32),
        pltpu.VMEM((NPAD,), jnp.float32),
        pltpu.VMEM_SHARED((NPAD, D), jnp.float32),
        pltpu.SemaphoreType.DMA,
    ],
    compiler_params=pltpu.CompilerParams(use_tc_tiling_on_sc=False,
                                         needs_layout_passes=False),
)
def _sc_segsum(u_hbm, dst_hbm, src_hbm, ztbl_hbm, zcnt_hbm,
               acc_out, cnt_out, dst_v, src_v, rows, cnt_v, acc_sh, sg):
    c = lax.axis_index("c")
    s = lax.axis_index("s")
    w = c * 16 + s
    pltpu.sync_copy(ztbl_hbm.at[pl.ds(s * STRIPE, STRIPE)],
                    acc_sh.at[pl.ds(s * STRIPE, STRIPE)])
    pltpu.sync_copy(zcnt_hbm, cnt_v)
    pltpu.sync_copy(dst_hbm.at[w], dst_v)
    pltpu.sync_copy(src_hbm.at[w], src_v)
    plsc.subcore_barrier()

    ones16 = jnp.ones((16,), jnp.float32)

    def body(j, carry):
        cp = pltpu.async_copy(u_hbm.at[dst_v.at[j]], rows, sg)
        for g in range(GRP):
            sv = src_v[j, pl.ds(g * 16, 16)]
            plsc.addupdate_scatter(cnt_v, [sv], ones16)
        cp.wait()
        pltpu.sync_copy(rows, acc_sh.at[src_v.at[j]], add=True)
        return carry

    lax.fori_loop(0, CH, body, 0)
    plsc.subcore_barrier()

    pltpu.sync_copy(acc_sh.at[pl.ds(s * STRIPE, STRIPE)],
                    acc_out.at[pl.ds(c * NPAD + s * STRIPE, STRIPE)])
    pltpu.sync_copy(cnt_v, cnt_out.at[w])


BLK = 1024


def _prep_body(x_ref, pos_ref, w2_ref, w3_ref, u_ref):
    u_ref[...] = (jnp.dot(x_ref[...], w2_ref[...],
                          preferred_element_type=jnp.float32)
                  + jnp.dot(pos_ref[...], w3_ref[...],
                            preferred_element_type=jnp.float32))


def _tc_prep(x, pos8, w2, w3p):
    full = lambda shape: pl.BlockSpec(shape, lambda i: (0,) * len(shape))
    return pl.pallas_call(
        _prep_body,
        grid=(NPAD // BLK,),
        in_specs=[
            pl.BlockSpec((BLK, D), lambda i: (i, 0)),
            pl.BlockSpec((BLK, 8), lambda i: (i, 0)),
            full((D, D)),
            full((8, D)),
        ],
        out_specs=pl.BlockSpec((BLK, D), lambda i: (i, 0)),
        out_shape=jax.ShapeDtypeStruct((NPAD, D), jnp.float32),
    )(x, pos8, w2, w3p)


def _combine_body(x_ref, u_ref, acc_ref, cnt_ref, w1_ref, wa1_ref, wa2_ref,
                  bmsg_ref, bagg_ref, out_ref):
    x = x_ref[...]
    acc = acc_ref[0] + acc_ref[1]
    cnt = jnp.sum(cnt_ref[...], axis=1)[:, None]
    maxc = jnp.maximum(cnt, 1.0)
    ind = (cnt > 0.0).astype(jnp.float32)
    aggr = (ind * (jnp.dot(x, w1_ref[...],
                           preferred_element_type=jnp.float32)
                   - u_ref[...] + bmsg_ref[...])
            + acc / maxc)
    out_ref[...] = (jnp.dot(x, wa1_ref[...],
                            preferred_element_type=jnp.float32)
                    + jnp.dot(aggr, wa2_ref[...],
                              preferred_element_type=jnp.float32)
                    + bagg_ref[...])


def _tc_combine(x, u, acc, cnt, w1, wa1, wa2, bmsg, bagg):
    full = lambda shape: pl.BlockSpec(shape, lambda i: (0,) * len(shape))
    return pl.pallas_call(
        _combine_body,
        grid=(NPAD // BLK,),
        in_specs=[
            pl.BlockSpec((BLK, D), lambda i: (i, 0)),
            pl.BlockSpec((BLK, D), lambda i: (i, 0)),
            pl.BlockSpec((2, BLK, D), lambda i: (0, i, 0)),
            pl.BlockSpec((BLK, NW), lambda i: (i, 0)),
            full((D, D)),
            full((D, D)),
            full((D, D)),
            full((1, D)),
            full((1, D)),
        ],
        out_specs=pl.BlockSpec((BLK, D), lambda i: (i, 0)),
        out_shape=jax.ShapeDtypeStruct((NPAD, D), jnp.float32),
    )(x, u, acc, cnt, w1, wa1, wa2, bmsg, bagg)


def kernel(x, edge_index, pos, W_msg, b_msg, W_agg, b_agg):
    src = edge_index[0].astype(jnp.int32)
    dst = edge_index[1].astype(jnp.int32)
    ppw = CH * CHUNK - E // NW
    pad_rows = N + (jnp.arange(NW * ppw, dtype=jnp.int32) % (NPAD - N))
    pad_rows = pad_rows.reshape(NW, ppw)
    src_p = jnp.concatenate([src.reshape(NW, E // NW), pad_rows],
                            axis=1).reshape(NW, CH, CHUNK)
    dst_p = jnp.concatenate([dst.reshape(NW, E // NW), pad_rows],
                            axis=1).reshape(NW, CH, CHUNK)

    x_pad = jnp.zeros((NPAD, D), jnp.float32).at[:N].set(x)
    pos8 = jnp.zeros((NPAD, 8), jnp.float32).at[:N, :P].set(pos)

    W1, W2, W3 = W_msg[:D], W_msg[D:2 * D], W_msg[2 * D:]
    w3p = jnp.concatenate([W3, jnp.zeros((8 - P, D), jnp.float32)], axis=0)

    u = _tc_prep(x_pad, pos8, W2, w3p)

    ztbl = jnp.zeros((NPAD, D), jnp.float32)
    zcnt = jnp.zeros((NPAD,), jnp.float32)
    acc, cnt = _sc_segsum(u, dst_p, src_p, ztbl, zcnt)
    acc = acc.reshape(2, NPAD, D)
    cnt = cnt.T

    out = _tc_combine(x_pad, u, acc, cnt, W1, W_agg[:D], W_agg[D:],
                      b_msg[None, :], b_agg[None, :])
    return out[:N]


# speedup vs baseline: 2.6202x; 1.0174x over previous
import functools

import jax
import jax.numpy as jnp
from jax import lax
from jax.experimental import pallas as pl
from jax.experimental.pallas import tpu as pltpu
from jax.experimental.pallas import tpu_sc as plsc

N = 10000
E = 320000
D = 128
P = 3

NPAD = 10240
NW = 32
CHUNK = 128
CH = 79
EPAD = NW * CH * CHUNK
STRIPE = NPAD // 16
GRP = CHUNK // 16

_mesh = plsc.VectorSubcoreMesh(core_axis_name="c", subcore_axis_name="s")


@functools.partial(
    pl.kernel,
    mesh=_mesh,
    out_type=(
        jax.ShapeDtypeStruct((2 * NPAD, D), jnp.float32),
        jax.ShapeDtypeStruct((NW, NPAD), jnp.float32),
    ),
    scratch_types=[
        pltpu.VMEM((CH, CHUNK), jnp.int32),
        pltpu.VMEM((CH, CHUNK), jnp.int32),
        pltpu.VMEM((CHUNK, D), jnp.float32),
        pltpu.VMEM((NPAD,), jnp.float32),
        pltpu.VMEM_SHARED((NPAD, D), jnp.float32),
        pltpu.SemaphoreType.DMA,
    ],
    compiler_params=pltpu.CompilerParams(use_tc_tiling_on_sc=False,
                                         needs_layout_passes=False),
)
def _sc_segsum(u_hbm, dst_hbm, src_hbm, ztbl_hbm, zcnt_hbm,
               acc_out, cnt_out, dst_v, src_v, rows, cnt_v, acc_sh, sg):
    c = lax.axis_index("c")
    s = lax.axis_index("s")
    w = c * 16 + s
    pltpu.sync_copy(ztbl_hbm.at[pl.ds(s * STRIPE, STRIPE)],
                    acc_sh.at[pl.ds(s * STRIPE, STRIPE)])
    pltpu.sync_copy(zcnt_hbm, cnt_v)
    pltpu.sync_copy(dst_hbm.at[w], dst_v)
    pltpu.sync_copy(src_hbm.at[w], src_v)
    plsc.subcore_barrier()

    ones16 = jnp.ones((16,), jnp.float32)

    def body(j, carry):
        cp = pltpu.async_copy(u_hbm.at[dst_v.at[j]], rows, sg)
        for g in range(GRP):
            sv = src_v[j, pl.ds(g * 16, 16)]
            plsc.addupdate_scatter(cnt_v, [sv], ones16)
        cp.wait()
        pltpu.sync_copy(rows, acc_sh.at[src_v.at[j]], add=True)
        return carry

    lax.fori_loop(0, CH, body, 0)
    plsc.subcore_barrier()

    pltpu.sync_copy(acc_sh.at[pl.ds(s * STRIPE, STRIPE)],
                    acc_out.at[pl.ds(c * NPAD + s * STRIPE, STRIPE)])
    pltpu.sync_copy(cnt_v, cnt_out.at[w])


BLK = 1024


def _prep_body(x_ref, pos_ref, w2_ref, w3_ref, u_ref):
    u_ref[...] = (jnp.dot(x_ref[...], w2_ref[...],
                          preferred_element_type=jnp.float32)
                  + jnp.dot(pos_ref[...], w3_ref[...],
                            preferred_element_type=jnp.float32))


def _tc_prep(x, pos8, w2, w3p):
    full = lambda shape: pl.BlockSpec(shape, lambda i: (0,) * len(shape))
    return pl.pallas_call(
        _prep_body,
        grid=(NPAD // BLK,),
        in_specs=[
            pl.BlockSpec((BLK, D), lambda i: (i, 0)),
            pl.BlockSpec((BLK, 8), lambda i: (i, 0)),
            full((D, D)),
            full((8, D)),
        ],
        out_specs=pl.BlockSpec((BLK, D), lambda i: (i, 0)),
        out_shape=jax.ShapeDtypeStruct((NPAD, D), jnp.float32),
    )(x, pos8, w2, w3p)


def _combine_body(x_ref, u_ref, acc_ref, cnt_ref, w1_ref, wa1_ref, wa2_ref,
                  bmsg_ref, bagg_ref, out_ref):
    x = x_ref[...]
    acc = acc_ref[0] + acc_ref[1]
    cnt = jnp.sum(cnt_ref[...], axis=0)[:, None]
    maxc = jnp.maximum(cnt, 1.0)
    ind = (cnt > 0.0).astype(jnp.float32)
    aggr = (ind * (jnp.dot(x, w1_ref[...],
                           preferred_element_type=jnp.float32)
                   - u_ref[...] + bmsg_ref[...])
            + acc / maxc)
    out_ref[...] = (jnp.dot(x, wa1_ref[...],
                            preferred_element_type=jnp.float32)
                    + jnp.dot(aggr, wa2_ref[...],
                              preferred_element_type=jnp.float32)
                    + bagg_ref[...])


def _tc_combine(x, u, acc, cnt, w1, wa1, wa2, bmsg, bagg):
    full = lambda shape: pl.BlockSpec(shape, lambda i: (0,) * len(shape))
    return pl.pallas_call(
        _combine_body,
        grid=(NPAD // BLK,),
        in_specs=[
            pl.BlockSpec((BLK, D), lambda i: (i, 0)),
            pl.BlockSpec((BLK, D), lambda i: (i, 0)),
            pl.BlockSpec((2, BLK, D), lambda i: (0, i, 0)),
            pl.BlockSpec((NW, BLK), lambda i: (0, i)),
            full((D, D)),
            full((D, D)),
            full((D, D)),
            full((1, D)),
            full((1, D)),
        ],
        out_specs=pl.BlockSpec((BLK, D), lambda i: (i, 0)),
        out_shape=jax.ShapeDtypeStruct((NPAD, D), jnp.float32),
    )(x, u, acc, cnt, w1, wa1, wa2, bmsg, bagg)


def kernel(x, edge_index, pos, W_msg, b_msg, W_agg, b_agg):
    src = edge_index[0].astype(jnp.int32)
    dst = edge_index[1].astype(jnp.int32)
    ppw = CH * CHUNK - E // NW
    pad_rows = N + (jnp.arange(NW * ppw, dtype=jnp.int32) % (NPAD - N))
    pad_rows = pad_rows.reshape(NW, ppw)
    src_p = jnp.concatenate([src.reshape(NW, E // NW), pad_rows],
                            axis=1).reshape(NW, CH, CHUNK)
    dst_p = jnp.concatenate([dst.reshape(NW, E // NW), pad_rows],
                            axis=1).reshape(NW, CH, CHUNK)

    x_pad = jnp.zeros((NPAD, D), jnp.float32).at[:N].set(x)
    pos8 = jnp.zeros((NPAD, 8), jnp.float32).at[:N, :P].set(pos)

    W1, W2, W3 = W_msg[:D], W_msg[D:2 * D], W_msg[2 * D:]
    w3p = jnp.concatenate([W3, jnp.zeros((8 - P, D), jnp.float32)], axis=0)

    u = _tc_prep(x_pad, pos8, W2, w3p)

    ztbl = jnp.zeros((NPAD, D), jnp.float32)
    zcnt = jnp.zeros((NPAD,), jnp.float32)
    acc, cnt = _sc_segsum(u, dst_p, src_p, ztbl, zcnt)
    acc = acc.reshape(2, NPAD, D)

    out = _tc_combine(x_pad, u, acc, cnt, W1, W_agg[:D], W_agg[D:],
                      b_msg[None, :], b_agg[None, :])
    return out[:N]


# speedup vs baseline: 2.6621x; 1.0160x over previous
import functools

import jax
import jax.numpy as jnp
from jax import lax
from jax.experimental import pallas as pl
from jax.experimental.pallas import tpu as pltpu
from jax.experimental.pallas import tpu_sc as plsc

N = 10000
E = 320000
D = 128
P = 3

NPAD = 10240
NW = 32
CHUNK = 128
CH = 79
EPAD = NW * CH * CHUNK
STRIPE = NPAD // 16
GRP = CHUNK // 16

_mesh = plsc.VectorSubcoreMesh(core_axis_name="c", subcore_axis_name="s")


@functools.partial(
    pl.kernel,
    mesh=_mesh,
    out_type=(
        jax.ShapeDtypeStruct((2 * NPAD, D), jnp.float32),
        jax.ShapeDtypeStruct((NW, NPAD), jnp.float32),
    ),
    scratch_types=[
        pltpu.VMEM((CH, CHUNK), jnp.int32),
        pltpu.VMEM((CH, CHUNK), jnp.int32),
        pltpu.VMEM((CHUNK, D), jnp.float32),
        pltpu.VMEM((NPAD,), jnp.float32),
        pltpu.VMEM_SHARED((NPAD, D), jnp.float32),
        pltpu.SemaphoreType.DMA,
    ],
    compiler_params=pltpu.CompilerParams(use_tc_tiling_on_sc=False,
                                         needs_layout_passes=False),
)
def _sc_segsum(u_hbm, dst_hbm, src_hbm, ztbl_hbm, zcnt_hbm,
               acc_out, cnt_out, dst_v, src_v, rows, cnt_v, acc_sh, sg):
    c = lax.axis_index("c")
    s = lax.axis_index("s")
    w = c * 16 + s
    pltpu.sync_copy(ztbl_hbm.at[pl.ds(s * STRIPE, STRIPE)],
                    acc_sh.at[pl.ds(s * STRIPE, STRIPE)])
    pltpu.sync_copy(zcnt_hbm, cnt_v)
    pltpu.sync_copy(dst_hbm.at[w], dst_v)
    pltpu.sync_copy(src_hbm.at[w], src_v)
    plsc.subcore_barrier()

    ones16 = jnp.ones((16,), jnp.float32)

    def body(j, carry):
        cp = pltpu.async_copy(u_hbm.at[dst_v.at[j]], rows, sg)
        for g in range(GRP):
            sv = src_v[j, pl.ds(g * 16, 16)]
            plsc.addupdate_scatter(cnt_v, [sv], ones16)
        cp.wait()
        pltpu.sync_copy(rows, acc_sh.at[src_v.at[j]], add=True)
        return carry

    lax.fori_loop(0, CH, body, 0)
    plsc.subcore_barrier()

    pltpu.sync_copy(acc_sh.at[pl.ds(s * STRIPE, STRIPE)],
                    acc_out.at[pl.ds(c * NPAD + s * STRIPE, STRIPE)])
    pltpu.sync_copy(cnt_v, cnt_out.at[w])


BLK = 2048


def _prep_body(x_ref, pos_ref, w2_ref, w3_ref, u_ref):
    u_ref[...] = (jnp.dot(x_ref[...], w2_ref[...],
                          preferred_element_type=jnp.float32)
                  + jnp.dot(pos_ref[...], w3_ref[...],
                            preferred_element_type=jnp.float32))


def _tc_prep(x, pos8, w2, w3p):
    full = lambda shape: pl.BlockSpec(shape, lambda i: (0,) * len(shape))
    return pl.pallas_call(
        _prep_body,
        grid=(NPAD // BLK,),
        in_specs=[
            pl.BlockSpec((BLK, D), lambda i: (i, 0)),
            pl.BlockSpec((BLK, 8), lambda i: (i, 0)),
            full((D, D)),
            full((8, D)),
        ],
        out_specs=pl.BlockSpec((BLK, D), lambda i: (i, 0)),
        out_shape=jax.ShapeDtypeStruct((NPAD, D), jnp.float32),
    )(x, pos8, w2, w3p)


def _combine_body(x_ref, u_ref, acc_ref, cnt_ref, w1_ref, wa1_ref, wa2_ref,
                  bmsg_ref, bagg_ref, out_ref):
    x = x_ref[...]
    acc = acc_ref[0] + acc_ref[1]
    cnt = jnp.sum(cnt_ref[...], axis=0)[:, None]
    maxc = jnp.maximum(cnt, 1.0)
    ind = (cnt > 0.0).astype(jnp.float32)
    aggr = (ind * (jnp.dot(x, w1_ref[...],
                           preferred_element_type=jnp.float32)
                   - u_ref[...] + bmsg_ref[...])
            + acc / maxc)
    out_ref[...] = (jnp.dot(x, wa1_ref[...],
                            preferred_element_type=jnp.float32)
                    + jnp.dot(aggr, wa2_ref[...],
                              preferred_element_type=jnp.float32)
                    + bagg_ref[...])


def _tc_combine(x, u, acc, cnt, w1, wa1, wa2, bmsg, bagg):
    full = lambda shape: pl.BlockSpec(shape, lambda i: (0,) * len(shape))
    return pl.pallas_call(
        _combine_body,
        grid=(NPAD // BLK,),
        in_specs=[
            pl.BlockSpec((BLK, D), lambda i: (i, 0)),
            pl.BlockSpec((BLK, D), lambda i: (i, 0)),
            pl.BlockSpec((2, BLK, D), lambda i: (0, i, 0)),
            pl.BlockSpec((NW, BLK), lambda i: (0, i)),
            full((D, D)),
            full((D, D)),
            full((D, D)),
            full((1, D)),
            full((1, D)),
        ],
        out_specs=pl.BlockSpec((BLK, D), lambda i: (i, 0)),
        out_shape=jax.ShapeDtypeStruct((NPAD, D), jnp.float32),
    )(x, u, acc, cnt, w1, wa1, wa2, bmsg, bagg)


def kernel(x, edge_index, pos, W_msg, b_msg, W_agg, b_agg):
    src = edge_index[0].astype(jnp.int32)
    dst = edge_index[1].astype(jnp.int32)
    ppw = CH * CHUNK - E // NW
    pad_rows = N + (jnp.arange(NW * ppw, dtype=jnp.int32) % (NPAD - N))
    pad_rows = pad_rows.reshape(NW, ppw)
    src_p = jnp.concatenate([src.reshape(NW, E // NW), pad_rows],
                            axis=1).reshape(NW, CH, CHUNK)
    dst_p = jnp.concatenate([dst.reshape(NW, E // NW), pad_rows],
                            axis=1).reshape(NW, CH, CHUNK)

    x_pad = jnp.zeros((NPAD, D), jnp.float32).at[:N].set(x)
    pos8 = jnp.zeros((NPAD, 8), jnp.float32).at[:N, :P].set(pos)

    W1, W2, W3 = W_msg[:D], W_msg[D:2 * D], W_msg[2 * D:]
    w3p = jnp.concatenate([W3, jnp.zeros((8 - P, D), jnp.float32)], axis=0)

    u = _tc_prep(x_pad, pos8, W2, w3p)

    ztbl = jnp.zeros((NPAD, D), jnp.float32)
    zcnt = jnp.zeros((NPAD,), jnp.float32)
    acc, cnt = _sc_segsum(u, dst_p, src_p, ztbl, zcnt)
    acc = acc.reshape(2, NPAD, D)

    out = _tc_combine(x_pad, u, acc, cnt, W1, W_agg[:D], W_agg[D:],
                      b_msg[None, :], b_agg[None, :])
    return out[:N]


# speedup vs baseline: 2.6993x; 1.0140x over previous
import functools

import jax
import jax.numpy as jnp
from jax import lax
from jax.experimental import pallas as pl
from jax.experimental.pallas import tpu as pltpu
from jax.experimental.pallas import tpu_sc as plsc

N = 10000
E = 320000
D = 128
P = 3

NPAD = 10240
NW = 32
CHUNK = 128
CH = 79
EPAD = NW * CH * CHUNK
STRIPE = NPAD // 16
GRP = CHUNK // 16

_mesh = plsc.VectorSubcoreMesh(core_axis_name="c", subcore_axis_name="s")


@functools.partial(
    pl.kernel,
    mesh=_mesh,
    out_type=(
        jax.ShapeDtypeStruct((2 * NPAD, D), jnp.float32),
        jax.ShapeDtypeStruct((NW, NPAD), jnp.float32),
    ),
    scratch_types=[
        pltpu.VMEM((CH, CHUNK), jnp.int32),
        pltpu.VMEM((CH, CHUNK), jnp.int32),
        pltpu.VMEM((CHUNK, D), jnp.float32),
        pltpu.VMEM((NPAD,), jnp.float32),
        pltpu.VMEM_SHARED((NPAD, D), jnp.float32),
        pltpu.SemaphoreType.DMA,
    ],
    compiler_params=pltpu.CompilerParams(use_tc_tiling_on_sc=False,
                                         needs_layout_passes=False),
)
def _sc_segsum(u_hbm, dst_hbm, src_hbm, ztbl_hbm, zcnt_hbm,
               acc_out, cnt_out, dst_v, src_v, rows, cnt_v, acc_sh, sg):
    c = lax.axis_index("c")
    s = lax.axis_index("s")
    w = c * 16 + s
    pltpu.sync_copy(ztbl_hbm.at[pl.ds(s * STRIPE, STRIPE)],
                    acc_sh.at[pl.ds(s * STRIPE, STRIPE)])
    pltpu.sync_copy(zcnt_hbm, cnt_v)
    pltpu.sync_copy(dst_hbm.at[w], dst_v)
    pltpu.sync_copy(src_hbm.at[w], src_v)
    plsc.subcore_barrier()

    ones16 = jnp.ones((16,), jnp.float32)

    def body(j, carry):
        cp = pltpu.async_copy(u_hbm.at[dst_v.at[j]], rows, sg)
        for g in range(GRP):
            sv = src_v[j, pl.ds(g * 16, 16)]
            plsc.addupdate_scatter(cnt_v, [sv], ones16)
        cp.wait()
        pltpu.sync_copy(rows, acc_sh.at[src_v.at[j]], add=True)
        return carry

    lax.fori_loop(0, CH, body, 0)
    plsc.subcore_barrier()

    pltpu.sync_copy(acc_sh.at[pl.ds(s * STRIPE, STRIPE)],
                    acc_out.at[pl.ds(c * NPAD + s * STRIPE, STRIPE)])
    pltpu.sync_copy(cnt_v, cnt_out.at[w])


BLK = 5120


def _prep_body(x_ref, pos_ref, w2_ref, w3_ref, u_ref):
    u_ref[...] = (jnp.dot(x_ref[...], w2_ref[...],
                          preferred_element_type=jnp.float32)
                  + jnp.dot(pos_ref[...], w3_ref[...],
                            preferred_element_type=jnp.float32))


def _tc_prep(x, pos8, w2, w3p):
    full = lambda shape: pl.BlockSpec(shape, lambda i: (0,) * len(shape))
    return pl.pallas_call(
        _prep_body,
        grid=(NPAD // BLK,),
        in_specs=[
            pl.BlockSpec((BLK, D), lambda i: (i, 0)),
            pl.BlockSpec((BLK, 8), lambda i: (i, 0)),
            full((D, D)),
            full((8, D)),
        ],
        out_specs=pl.BlockSpec((BLK, D), lambda i: (i, 0)),
        out_shape=jax.ShapeDtypeStruct((NPAD, D), jnp.float32),
    )(x, pos8, w2, w3p)


def _combine_body(x_ref, u_ref, acc_ref, cnt_ref, w1_ref, wa1_ref, wa2_ref,
                  bmsg_ref, bagg_ref, out_ref):
    x = x_ref[...]
    acc = acc_ref[0] + acc_ref[1]
    cnt = jnp.sum(cnt_ref[...], axis=0)[:, None]
    maxc = jnp.maximum(cnt, 1.0)
    ind = (cnt > 0.0).astype(jnp.float32)
    aggr = (ind * (jnp.dot(x, w1_ref[...],
                           preferred_element_type=jnp.float32)
                   - u_ref[...] + bmsg_ref[...])
            + acc / maxc)
    out_ref[...] = (jnp.dot(x, wa1_ref[...],
                            preferred_element_type=jnp.float32)
                    + jnp.dot(aggr, wa2_ref[...],
                              preferred_element_type=jnp.float32)
                    + bagg_ref[...])


def _tc_combine(x, u, acc, cnt, w1, wa1, wa2, bmsg, bagg):
    full = lambda shape: pl.BlockSpec(shape, lambda i: (0,) * len(shape))
    return pl.pallas_call(
        _combine_body,
        grid=(NPAD // BLK,),
        in_specs=[
            pl.BlockSpec((BLK, D), lambda i: (i, 0)),
            pl.BlockSpec((BLK, D), lambda i: (i, 0)),
            pl.BlockSpec((2, BLK, D), lambda i: (0, i, 0)),
            pl.BlockSpec((NW, BLK), lambda i: (0, i)),
            full((D, D)),
            full((D, D)),
            full((D, D)),
            full((1, D)),
            full((1, D)),
        ],
        out_specs=pl.BlockSpec((BLK, D), lambda i: (i, 0)),
        out_shape=jax.ShapeDtypeStruct((NPAD, D), jnp.float32),
    )(x, u, acc, cnt, w1, wa1, wa2, bmsg, bagg)


def kernel(x, edge_index, pos, W_msg, b_msg, W_agg, b_agg):
    src = edge_index[0].astype(jnp.int32)
    dst = edge_index[1].astype(jnp.int32)
    ppw = CH * CHUNK - E // NW
    pad_rows = N + (jnp.arange(NW * ppw, dtype=jnp.int32) % (NPAD - N))
    pad_rows = pad_rows.reshape(NW, ppw)
    src_p = jnp.concatenate([src.reshape(NW, E // NW), pad_rows],
                            axis=1).reshape(NW, CH, CHUNK)
    dst_p = jnp.concatenate([dst.reshape(NW, E // NW), pad_rows],
                            axis=1).reshape(NW, CH, CHUNK)

    x_pad = jnp.zeros((NPAD, D), jnp.float32).at[:N].set(x)
    pos8 = jnp.zeros((NPAD, 8), jnp.float32).at[:N, :P].set(pos)

    W1, W2, W3 = W_msg[:D], W_msg[D:2 * D], W_msg[2 * D:]
    w3p = jnp.concatenate([W3, jnp.zeros((8 - P, D), jnp.float32)], axis=0)

    u = _tc_prep(x_pad, pos8, W2, w3p)

    ztbl = jnp.zeros((NPAD, D), jnp.float32)
    zcnt = jnp.zeros((NPAD,), jnp.float32)
    acc, cnt = _sc_segsum(u, dst_p, src_p, ztbl, zcnt)
    acc = acc.reshape(2, NPAD, D)

    out = _tc_combine(x_pad, u, acc, cnt, W1, W_agg[:D], W_agg[D:],
                      b_msg[None, :], b_agg[None, :])
    return out[:N]
